# baseline (speedup 1.0000x reference)
import jax
import jax.numpy as jnp
from jax import lax
from jax.experimental import pallas as pl
from jax.experimental.pallas import tpu as pltpu
from jax.experimental.pallas import tpu_sc as plsc

N_NODES = 10000
DIM = 128
NEG_SLOPE = 0.2
SQD = 1.0 / (2.0 ** 0.5)

NC, NS, L = 2, 16, 16
NW = NC * NS
CH = 128
GRP = 2
NGRP = 40
G0 = 48
G1 = 32
GMAX = max(G0, G1)
NRING = 4
E_PAD = NS * (G0 + G1) * GRP * CH
N_PAD = 10240
RPT = N_PAD // NS


def _af_body(w_ref, x_ref, out_ref):
    out_ref[...] = lax.dot_general(
        w_ref[...], x_ref[...], (((1,), (1,)), ((), ())),
        preferred_element_type=jnp.float32)


def _tc_attention_features(x_pad, w8):
    return pl.pallas_call(
        _af_body,
        out_shape=jax.ShapeDtypeStruct((8, N_PAD), jnp.float32),
    )(w8, x_pad)


def _sc_w_body(af_hbm, row_hbm, col_hbm, mask_hbm, w_hbm,
               ar0_v, ar1_v, ac0_v, ac1_v, row_v, col_v, mask_v, w_v):
    c = lax.axis_index("c")
    s = lax.axis_index("s")

    pltpu.sync_copy(af_hbm.at[0], ar0_v)
    pltpu.sync_copy(af_hbm.at[1], ar1_v)
    pltpu.sync_copy(af_hbm.at[2], ac0_v)
    pltpu.sync_copy(af_hbm.at[3], ac1_v)

    def run(ng, base):
        pltpu.sync_copy(row_hbm.at[pl.ds(base, ng)], row_v.at[pl.ds(0, ng)])
        pltpu.sync_copy(col_hbm.at[pl.ds(base, ng)], col_v.at[pl.ds(0, ng)])
        pltpu.sync_copy(mask_hbm.at[pl.ds(base, ng)], mask_v.at[pl.ds(0, ng)])

        def group(g, _):
            for j in range(GRP):
                for g16 in range(CH // L):
                    sl = pl.ds(g16 * L, L)
                    r16 = row_v[g, j, sl]
                    c16 = col_v[g, j, sl]
                    m16 = mask_v[g, j, sl]
                    dot = (plsc.load_gather(ar0_v, [r16]) * plsc.load_gather(ac0_v, [c16])
                           + plsc.load_gather(ar1_v, [r16]) * plsc.load_gather(ac1_v, [c16]))
                    dot = dot * SQD
                    logit = jnp.where(dot >= 0, dot, NEG_SLOPE * dot)
                    w_v[g, j, sl] = m16 * jnp.exp(logit)
            return 0

        lax.fori_loop(0, ng, group, 0)
        pltpu.sync_copy(w_v.at[pl.ds(0, ng)], w_hbm.at[pl.ds(base, ng)])

    @pl.when(c == 0)
    def _():
        run(G0, s * G0)

    @pl.when(c == 1)
    def _():
        run(G1, NS * G0 + s * G1)


def _sc_weights(af_t, row2, col2, mask2):
    mesh = plsc.VectorSubcoreMesh(core_axis_name="c", subcore_axis_name="s",
                                  num_cores=NC, num_subcores=NS)
    f = pl.kernel(
        _sc_w_body,
        out_type=jax.ShapeDtypeStruct((NS * (G0 + G1), GRP, CH), jnp.float32),
        mesh=mesh,
        compiler_params=pltpu.CompilerParams(needs_layout_passes=False),
        scratch_types=(
            pltpu.VMEM((N_PAD,), jnp.float32),
            pltpu.VMEM((N_PAD,), jnp.float32),
            pltpu.VMEM((N_PAD,), jnp.float32),
            pltpu.VMEM((N_PAD,), jnp.float32),
            pltpu.VMEM((GMAX, GRP, CH), jnp.int32),
            pltpu.VMEM((GMAX, GRP, CH), jnp.int32),
            pltpu.VMEM((GMAX, GRP, CH), jnp.float32),
            pltpu.VMEM((GMAX, GRP, CH), jnp.float32),
        ),
    )
    return f(af_t, row2, col2, mask2)


def _sc_agg_body(x_hbm, row_hbm, col_hbm, w_hbm,
                 po_hbm, pd_hbm,
                 ridx, cidx, wk, rows0, rows1, den_t,
                 out_sp,
                 sem_i0, sem_i1, sem_i2, sem_i3, sem_g0, sem_g1,
                 sem_s0, sem_s1):
    c = lax.axis_index("c")
    s = lax.axis_index("s")

    rows_b = (rows0, rows1)
    sem_i = (sem_i0, sem_i1, sem_i2, sem_i3)
    sem_g = (sem_g0, sem_g1)
    sem_s = (sem_s0, sem_s1)

    def idx_copies(base, g, r):
        src = base + g
        return ((row_hbm.at[src], ridx.at[r]),
                (col_hbm.at[src], cidx.at[r]),
                (w_hbm.at[src], wk.at[r]))

    def issue_idx(base, g, r):
        for src, dst in idx_copies(base, g, r):
            pltpu.async_copy(src, dst, sem_i[r])

    def wait_idx(base, g, r):
        for src, dst in idx_copies(base, g, r):
            pltpu.make_async_copy(src, dst, sem_i[r]).wait()

    def gather_copy(r, j):
        return (x_hbm.at[cidx.at[r, j]], rows_b[j])

    def issue_gather(r, j):
        src, dst = gather_copy(r, j)
        pltpu.async_copy(src, dst, sem_g[j])

    def wait_gather(r, j):
        src, dst = gather_copy(r, j)
        pltpu.make_async_copy(src, dst, sem_g[j]).wait()

    def scatter_copy(r, j):
        return (rows_b[j], out_sp.at[ridx.at[r, j]])

    def issue_scatter(r, j):
        src, dst = scatter_copy(r, j)
        pltpu.async_copy(src, dst, sem_s[j], add=True)

    def wait_scatter(r, j):
        src, dst = scatter_copy(r, j)
        pltpu.make_async_copy(src, dst, sem_s[j]).wait()

    def accum_den(r, j):
        for g16 in range(CH // L):
            sl = pl.ds(g16 * L, L)
            plsc.addupdate_scatter(den_t, [ridx[r, j, sl]], wk[r, j, sl])

    def scale_rows(r, j):
        rows = rows_b[j]
        rr = jnp.full((L,), r, jnp.int32)
        jj = jnp.full((L,), j, jnp.int32)
        zz = jnp.zeros((L,), jnp.int32)

        def body(e0, _):
            for u in range(4):
                e = e0 * 4 + u
                wsp = plsc.load_gather(wk, [rr, jj, zz + e])
                for d in range(DIM // L):
                    sl = pl.ds(d * L, L)
                    rows[e, sl] = rows[e, sl] * wsp
            return 0

        lax.fori_loop(0, CH // 4, body, 0)

    def zden(e, _):
        den_t[pl.ds(e * L, L)] = jnp.zeros((L,), jnp.float32)
        return 0

    lax.fori_loop(0, N_PAD // L, zden, 0)

    def zrows(e, _):
        for d in range(DIM // L):
            rows0[e, pl.ds(d * L, L)] = jnp.zeros((L,), jnp.float32)
        return 0

    lax.fori_loop(0, CH, zrows, 0)
    for k in range(RPT // CH):
        pltpu.sync_copy(rows0, out_sp.at[pl.ds(s * RPT + k * CH, CH)])
    plsc.subcore_barrier()

    def pipeline(ng, base):
        for src, dst in idx_copies(base, 0, 0):
            pltpu.sync_copy(src, dst)
        issue_idx(base, 1, 1)
        issue_idx(base, 2, 2)
        issue_gather(0, 0)

        def outer(og, _):
            for gb in range(NRING):
                g = NRING * og + gb

                i0 = 2 * g
                wait_gather(gb, 0)

                @pl.when(i0 >= 1)
                def _():
                    wait_scatter((gb + 3) % NRING, 1)

                @pl.when(g + 3 < ng)
                def _():
                    issue_idx(base, g + 3, (gb + 3) % NRING)

                issue_gather(gb, 1)
                scale_rows(gb, 0)
                accum_den(gb, 0)
                issue_scatter(gb, 0)

                wait_gather(gb, 1)
                wait_scatter(gb, 0)

                @pl.when(g + 1 < ng)
                def _():
                    wait_idx(base, g + 1, (gb + 1) % NRING)
                    issue_gather((gb + 1) % NRING, 0)

                scale_rows(gb, 1)
                accum_den(gb, 1)
                issue_scatter(gb, 1)
            return 0

        lax.fori_loop(0, ng // NRING, outer, 0)
        wait_scatter((ng - 1) % NRING, 1)

    @pl.when(c == 0)
    def _():
        pipeline(G0, s * G0)

    @pl.when(c == 1)
    def _():
        pipeline(G1, NS * G0 + s * G1)

    plsc.subcore_barrier()

    sl = pl.ds(s * RPT, RPT)
    pltpu.sync_copy(out_sp.at[sl], po_hbm.at[c, sl])
    pltpu.sync_copy(den_t, pd_hbm.at[c, s])


def _sc_aggregate(x_pad, row2, col2, w2):
    mesh = plsc.VectorSubcoreMesh(core_axis_name="c", subcore_axis_name="s",
                                  num_cores=NC, num_subcores=NS)
    f = pl.kernel(
        _sc_agg_body,
        out_type=(jax.ShapeDtypeStruct((NC, N_PAD, DIM), jnp.float32),
                  jax.ShapeDtypeStruct((NC, NS, N_PAD), jnp.float32)),
        mesh=mesh,
        compiler_params=pltpu.CompilerParams(needs_layout_passes=False),
        scratch_types=(
            pltpu.VMEM((NRING, GRP, CH), jnp.int32),
            pltpu.VMEM((NRING, GRP, CH), jnp.int32),
            pltpu.VMEM((NRING, GRP, CH), jnp.float32),
            pltpu.VMEM((CH, DIM), jnp.float32),
            pltpu.VMEM((CH, DIM), jnp.float32),
            pltpu.VMEM((N_PAD,), jnp.float32),
            pltpu.VMEM_SHARED((N_PAD, DIM), jnp.float32),
            pltpu.SemaphoreType.DMA,
            pltpu.SemaphoreType.DMA,
            pltpu.SemaphoreType.DMA,
            pltpu.SemaphoreType.DMA,
            pltpu.SemaphoreType.DMA,
            pltpu.SemaphoreType.DMA,
            pltpu.SemaphoreType.DMA,
            pltpu.SemaphoreType.DMA,
        ),
    )
    return f(x_pad, row2, col2, w2)


def _finish_body(po_ref, pd_ref, wx_ref, b_ref, out_ref):
    acc = po_ref[0] + po_ref[1]
    den = jnp.sum(pd_ref[...], axis=(0, 1))
    inv = 1.0 / (den + 1e-15)
    scaled = acc * inv[:, None]
    out_ref[...] = lax.dot_general(
        scaled, wx_ref[...], (((1,), (1,)), ((), ())),
        preferred_element_type=jnp.float32) + b_ref[...]


def _tc_finish(po, pd, w_x, b2):
    blk = 1280
    grid = N_PAD // blk
    return pl.pallas_call(
        _finish_body,
        grid=(grid,),
        in_specs=[
            pl.BlockSpec((NC, blk, DIM), lambda i: (0, i, 0)),
            pl.BlockSpec((NC, NS, blk), lambda i: (0, 0, i)),
            pl.BlockSpec((DIM, DIM), lambda i: (0, 0)),
            pl.BlockSpec((1, DIM), lambda i: (0, 0)),
        ],
        out_specs=pl.BlockSpec((blk, DIM), lambda i: (i, 0)),
        out_shape=jax.ShapeDtypeStruct((N_PAD, DIM), jnp.float32),
    )(po, pd, w_x, b2)


def kernel(x, edge_index, mask_values, W_row, W_col, W_x, b_x):
    x = x.astype(jnp.float32)
    n, dim = x.shape
    e = edge_index.shape[1]
    row = edge_index[0].astype(jnp.int32)
    col = edge_index[1].astype(jnp.int32)
    pad = E_PAD - e
    row2 = jnp.concatenate([row, jnp.zeros((pad,), jnp.int32)]).reshape(NS * (G0 + G1), GRP, CH)
    col2 = jnp.concatenate([col, jnp.zeros((pad,), jnp.int32)]).reshape(NS * (G0 + G1), GRP, CH)
    mask2 = jnp.concatenate(
        [mask_values.astype(jnp.float32), jnp.zeros((pad,), jnp.float32)]
    ).reshape(NS * (G0 + G1), GRP, CH)
    x_pad = jnp.pad(x, ((0, N_PAD - n), (0, 0)))
    w8 = jnp.pad(jnp.concatenate([W_row, W_col], axis=0), ((0, 4), (0, 0)))

    af_t = _tc_attention_features(x_pad, w8)
    w2 = _sc_weights(af_t, row2, col2, mask2)
    po, pd = _sc_aggregate(x_pad, row2, col2, w2)
    out = _tc_finish(po, pd, W_x, b_x.reshape(1, DIM))
    return out[:n]


# speedup vs baseline: 1.0154x; 1.0154x over previous
import jax
import jax.numpy as jnp
from jax import lax
from jax.experimental import pallas as pl
from jax.experimental.pallas import tpu as pltpu
from jax.experimental.pallas import tpu_sc as plsc

N_NODES = 10000
DIM = 128
NEG_SLOPE = 0.2
SQD = 1.0 / (2.0 ** 0.5)

NC, NS, L = 2, 16, 16
NW = NC * NS
CH = 128
GRP = 2
NGRP = 40
G0 = 56
G1 = 24
GMAX = max(G0, G1)
NRING = 4
E_PAD = NS * (G0 + G1) * GRP * CH
N_PAD = 10240
RPT = N_PAD // NS


def _af_body(w_ref, x_ref, out_ref):
    out_ref[...] = lax.dot_general(
        w_ref[...], x_ref[...], (((1,), (1,)), ((), ())),
        preferred_element_type=jnp.float32)


def _tc_attention_features(x_pad, w8):
    return pl.pallas_call(
        _af_body,
        out_shape=jax.ShapeDtypeStruct((8, N_PAD), jnp.float32),
    )(w8, x_pad)


def _sc_w_body(af_hbm, row_hbm, col_hbm, mask_hbm, w_hbm,
               ar0_v, ar1_v, ac0_v, ac1_v, row_v, col_v, mask_v, w_v):
    c = lax.axis_index("c")
    s = lax.axis_index("s")

    pltpu.sync_copy(af_hbm.at[0], ar0_v)
    pltpu.sync_copy(af_hbm.at[1], ar1_v)
    pltpu.sync_copy(af_hbm.at[2], ac0_v)
    pltpu.sync_copy(af_hbm.at[3], ac1_v)

    def run(ng, base):
        pltpu.sync_copy(row_hbm.at[pl.ds(base, ng)], row_v.at[pl.ds(0, ng)])
        pltpu.sync_copy(col_hbm.at[pl.ds(base, ng)], col_v.at[pl.ds(0, ng)])
        pltpu.sync_copy(mask_hbm.at[pl.ds(base, ng)], mask_v.at[pl.ds(0, ng)])

        def group(g, _):
            for j in range(GRP):
                for g16 in range(CH // L):
                    sl = pl.ds(g16 * L, L)
                    r16 = row_v[g, j, sl]
                    c16 = col_v[g, j, sl]
                    m16 = mask_v[g, j, sl]
                    dot = (plsc.load_gather(ar0_v, [r16]) * plsc.load_gather(ac0_v, [c16])
                           + plsc.load_gather(ar1_v, [r16]) * plsc.load_gather(ac1_v, [c16]))
                    dot = dot * SQD
                    logit = jnp.where(dot >= 0, dot, NEG_SLOPE * dot)
                    w_v[g, j, sl] = m16 * jnp.exp(logit)
            return 0

        lax.fori_loop(0, ng, group, 0)
        pltpu.sync_copy(w_v.at[pl.ds(0, ng)], w_hbm.at[pl.ds(base, ng)])

    @pl.when(c == 0)
    def _():
        run(G0, s * G0)

    @pl.when(c == 1)
    def _():
        run(G1, NS * G0 + s * G1)


def _sc_weights(af_t, row2, col2, mask2):
    mesh = plsc.VectorSubcoreMesh(core_axis_name="c", subcore_axis_name="s",
                                  num_cores=NC, num_subcores=NS)
    f = pl.kernel(
        _sc_w_body,
        out_type=jax.ShapeDtypeStruct((NS * (G0 + G1), GRP, CH), jnp.float32),
        mesh=mesh,
        compiler_params=pltpu.CompilerParams(needs_layout_passes=False),
        scratch_types=(
            pltpu.VMEM((N_PAD,), jnp.float32),
            pltpu.VMEM((N_PAD,), jnp.float32),
            pltpu.VMEM((N_PAD,), jnp.float32),
            pltpu.VMEM((N_PAD,), jnp.float32),
            pltpu.VMEM((GMAX, GRP, CH), jnp.int32),
            pltpu.VMEM((GMAX, GRP, CH), jnp.int32),
            pltpu.VMEM((GMAX, GRP, CH), jnp.float32),
            pltpu.VMEM((GMAX, GRP, CH), jnp.float32),
        ),
    )
    return f(af_t, row2, col2, mask2)


def _sc_agg_body(x_hbm, row_hbm, col_hbm, w_hbm,
                 po_hbm, pd_hbm,
                 ridx, cidx, wk, rows0, rows1, den_t,
                 out_sp,
                 sem_i0, sem_i1, sem_i2, sem_i3, sem_g0, sem_g1,
                 sem_s0, sem_s1):
    c = lax.axis_index("c")
    s = lax.axis_index("s")

    rows_b = (rows0, rows1)
    sem_i = (sem_i0, sem_i1, sem_i2, sem_i3)
    sem_g = (sem_g0, sem_g1)
    sem_s = (sem_s0, sem_s1)

    def idx_copies(base, g, r):
        src = base + g
        return ((row_hbm.at[src], ridx.at[r]),
                (col_hbm.at[src], cidx.at[r]),
                (w_hbm.at[src], wk.at[r]))

    def issue_idx(base, g, r):
        for src, dst in idx_copies(base, g, r):
            pltpu.async_copy(src, dst, sem_i[r])

    def wait_idx(base, g, r):
        for src, dst in idx_copies(base, g, r):
            pltpu.make_async_copy(src, dst, sem_i[r]).wait()

    def gather_copy(r, j):
        return (x_hbm.at[cidx.at[r, j]], rows_b[j])

    def issue_gather(r, j):
        src, dst = gather_copy(r, j)
        pltpu.async_copy(src, dst, sem_g[j])

    def wait_gather(r, j):
        src, dst = gather_copy(r, j)
        pltpu.make_async_copy(src, dst, sem_g[j]).wait()

    def scatter_copy(r, j):
        return (rows_b[j], out_sp.at[ridx.at[r, j]])

    def issue_scatter(r, j):
        src, dst = scatter_copy(r, j)
        pltpu.async_copy(src, dst, sem_s[j], add=True)

    def wait_scatter(r, j):
        src, dst = scatter_copy(r, j)
        pltpu.make_async_copy(src, dst, sem_s[j]).wait()

    def accum_den(r, j):
        for g16 in range(CH // L):
            sl = pl.ds(g16 * L, L)
            plsc.addupdate_scatter(den_t, [ridx[r, j, sl]], wk[r, j, sl])

    def scale_rows(r, j):
        rows = rows_b[j]
        rr = jnp.full((L,), r, jnp.int32)
        jj = jnp.full((L,), j, jnp.int32)
        zz = jnp.zeros((L,), jnp.int32)

        def body(e0, _):
            for u in range(4):
                e = e0 * 4 + u
                wsp = plsc.load_gather(wk, [rr, jj, zz + e])
                for d in range(DIM // L):
                    sl = pl.ds(d * L, L)
                    rows[e, sl] = rows[e, sl] * wsp
            return 0

        lax.fori_loop(0, CH // 4, body, 0)

    def zden(e, _):
        den_t[pl.ds(e * L, L)] = jnp.zeros((L,), jnp.float32)
        return 0

    lax.fori_loop(0, N_PAD // L, zden, 0)

    def zrows(e, _):
        for d in range(DIM // L):
            rows0[e, pl.ds(d * L, L)] = jnp.zeros((L,), jnp.float32)
        return 0

    lax.fori_loop(0, CH, zrows, 0)
    for k in range(RPT // CH):
        pltpu.sync_copy(rows0, out_sp.at[pl.ds(s * RPT + k * CH, CH)])
    plsc.subcore_barrier()

    def pipeline(ng, base):
        for src, dst in idx_copies(base, 0, 0):
            pltpu.sync_copy(src, dst)
        issue_idx(base, 1, 1)
        issue_idx(base, 2, 2)
        issue_gather(0, 0)

        def outer(og, _):
            for gb in range(NRING):
                g = NRING * og + gb

                i0 = 2 * g
                wait_gather(gb, 0)

                @pl.when(i0 >= 1)
                def _():
                    wait_scatter((gb + 3) % NRING, 1)

                @pl.when(g + 3 < ng)
                def _():
                    issue_idx(base, g + 3, (gb + 3) % NRING)

                issue_gather(gb, 1)
                scale_rows(gb, 0)
                accum_den(gb, 0)
                issue_scatter(gb, 0)

                wait_gather(gb, 1)
                wait_scatter(gb, 0)

                @pl.when(g + 1 < ng)
                def _():
                    wait_idx(base, g + 1, (gb + 1) % NRING)
                    issue_gather((gb + 1) % NRING, 0)

                scale_rows(gb, 1)
                accum_den(gb, 1)
                issue_scatter(gb, 1)
            return 0

        lax.fori_loop(0, ng // NRING, outer, 0)
        wait_scatter((ng - 1) % NRING, 1)

    @pl.when(c == 0)
    def _():
        pipeline(G0, s * G0)

    @pl.when(c == 1)
    def _():
        pipeline(G1, NS * G0 + s * G1)

    plsc.subcore_barrier()

    sl = pl.ds(s * RPT, RPT)
    pltpu.sync_copy(out_sp.at[sl], po_hbm.at[c, sl])
    pltpu.sync_copy(den_t, pd_hbm.at[c, s])


def _sc_aggregate(x_pad, row2, col2, w2):
    mesh = plsc.VectorSubcoreMesh(core_axis_name="c", subcore_axis_name="s",
                                  num_cores=NC, num_subcores=NS)
    f = pl.kernel(
        _sc_agg_body,
        out_type=(jax.ShapeDtypeStruct((NC, N_PAD, DIM), jnp.float32),
                  jax.ShapeDtypeStruct((NC, NS, N_PAD), jnp.float32)),
        mesh=mesh,
        compiler_params=pltpu.CompilerParams(needs_layout_passes=False),
        scratch_types=(
            pltpu.VMEM((NRING, GRP, CH), jnp.int32),
            pltpu.VMEM((NRING, GRP, CH), jnp.int32),
            pltpu.VMEM((NRING, GRP, CH), jnp.float32),
            pltpu.VMEM((CH, DIM), jnp.float32),
            pltpu.VMEM((CH, DIM), jnp.float32),
            pltpu.VMEM((N_PAD,), jnp.float32),
            pltpu.VMEM_SHARED((N_PAD, DIM), jnp.float32),
            pltpu.SemaphoreType.DMA,
            pltpu.SemaphoreType.DMA,
            pltpu.SemaphoreType.DMA,
            pltpu.SemaphoreType.DMA,
            pltpu.SemaphoreType.DMA,
            pltpu.SemaphoreType.DMA,
            pltpu.SemaphoreType.DMA,
            pltpu.SemaphoreType.DMA,
        ),
    )
    return f(x_pad, row2, col2, w2)


def _finish_body(po_ref, pd_ref, wx_ref, b_ref, out_ref):
    acc = po_ref[0] + po_ref[1]
    den = jnp.sum(pd_ref[...], axis=(0, 1))
    inv = 1.0 / (den + 1e-15)
    scaled = acc * inv[:, None]
    out_ref[...] = lax.dot_general(
        scaled, wx_ref[...], (((1,), (1,)), ((), ())),
        preferred_element_type=jnp.float32) + b_ref[...]


def _tc_finish(po, pd, w_x, b2):
    blk = 1280
    grid = N_PAD // blk
    return pl.pallas_call(
        _finish_body,
        grid=(grid,),
        in_specs=[
            pl.BlockSpec((NC, blk, DIM), lambda i: (0, i, 0)),
            pl.BlockSpec((NC, NS, blk), lambda i: (0, 0, i)),
            pl.BlockSpec((DIM, DIM), lambda i: (0, 0)),
            pl.BlockSpec((1, DIM), lambda i: (0, 0)),
        ],
        out_specs=pl.BlockSpec((blk, DIM), lambda i: (i, 0)),
        out_shape=jax.ShapeDtypeStruct((N_PAD, DIM), jnp.float32),
    )(po, pd, w_x, b2)


def kernel(x, edge_index, mask_values, W_row, W_col, W_x, b_x):
    x = x.astype(jnp.float32)
    n, dim = x.shape
    e = edge_index.shape[1]
    row = edge_index[0].astype(jnp.int32)
    col = edge_index[1].astype(jnp.int32)
    pad = E_PAD - e
    row2 = jnp.concatenate([row, jnp.zeros((pad,), jnp.int32)]).reshape(NS * (G0 + G1), GRP, CH)
    col2 = jnp.concatenate([col, jnp.zeros((pad,), jnp.int32)]).reshape(NS * (G0 + G1), GRP, CH)
    mask2 = jnp.concatenate(
        [mask_values.astype(jnp.float32), jnp.zeros((pad,), jnp.float32)]
    ).reshape(NS * (G0 + G1), GRP, CH)
    x_pad = jnp.pad(x, ((0, N_PAD - n), (0, 0)))
    w8 = jnp.pad(jnp.concatenate([W_row, W_col], axis=0), ((0, 4), (0, 0)))

    af_t = _tc_attention_features(x_pad, w8)
    w2 = _sc_weights(af_t, row2, col2, mask2)
    po, pd = _sc_aggregate(x_pad, row2, col2, w2)
    out = _tc_finish(po, pd, W_x, b_x.reshape(1, DIM))
    return out[:n]
